# baseline (speedup 1.0000x reference)
import functools

import jax
import jax.numpy as jnp
from jax import lax
from jax.experimental import pallas as pl
from jax.experimental.pallas import tpu as pltpu
from jax.experimental.pallas import tpu_sc as plsc

VOCAB = 1_000_000
EMBED_DIM = 32
BATCH = 16384

_NC = 2
_NS = 16
_NW = _NC * _NS
_B_PER_W = BATCH // _NW
_CHUNK = 64
_NCHUNK = _B_PER_W // _CHUNK
_PACK = 128 // EMBED_DIM

_mesh = plsc.VectorSubcoreMesh(core_axis_name="c", subcore_axis_name="s")


@functools.partial(
    pl.kernel,
    mesh=_mesh,
    out_type=jax.ShapeDtypeStruct((BATCH, EMBED_DIM), jnp.float32),
    scratch_types=[
        pltpu.VMEM((_B_PER_W,), jnp.int32),
        pltpu.VMEM((_CHUNK,), jnp.int32),
        pltpu.VMEM((_CHUNK, 128), jnp.float32),
        pltpu.VMEM((_CHUNK, EMBED_DIM), jnp.float32),
        pltpu.SemaphoreType.DMA,
    ],
    compiler_params=pltpu.CompilerParams(needs_layout_passes=False),
)
def _gather_kernel(idx_hbm, table_hbm, out_hbm, idx_v, gid_v, rows_v, outc_v, sem):
    wid = lax.axis_index("s") * _NC + lax.axis_index("c")
    base = wid * _B_PER_W
    pltpu.sync_copy(idx_hbm.at[pl.ds(base, _B_PER_W)], idx_v)
    lanes = lax.iota(jnp.int32, 16)

    for c in range(_NCHUNK):
        for g in range(_CHUNK // 16):
            gid_v[pl.ds(g * 16, 16)] = idx_v[pl.ds(c * _CHUNK + g * 16, 16)] >> 2
        pltpu.async_copy(table_hbm.at[gid_v], rows_v, sem).wait()
        for g in range(_CHUNK // 16):
            sub_v = (idx_v[pl.ds(c * _CHUNK + g * 16, 16)] & (_PACK - 1)) * EMBED_DIM
            rows16 = lax.iota(jnp.int32, 16) + g * 16
            for col in range(EMBED_DIM):
                vals = plsc.load_gather(rows_v, [rows16, sub_v + col])
                plsc.store_scatter(outc_v, [rows16, jnp.full((16,), col, jnp.int32)], vals)
        pltpu.sync_copy(outc_v, out_hbm.at[pl.ds(base + c * _CHUNK, _CHUNK)])


def kernel(indices, table):
    tbl2 = table.reshape(VOCAB // _PACK, 128)
    return _gather_kernel(indices.astype(jnp.int32), tbl2)
